# baseline (speedup 1.0000x reference)
import functools

import jax
import jax.numpy as jnp
from jax import lax
from jax.experimental import pallas as pl
from jax.experimental.pallas import tpu as pltpu
from jax.experimental.pallas import tpu_sc as plsc

EMB_D = 128
_info = plsc.get_sparse_core_info()
_NC, _NS = _info.num_cores, _info.num_subcores
_NW = _NC * _NS

_CHUNK = 400
_SUB = 128


def _make_gather(n_total: int):
    assert n_total % (_NW * _CHUNK) == 0
    bpw = n_total // _NW
    n_chunks = bpw // _CHUNK
    assert n_chunks >= 4 and n_chunks % 2 == 0
    subs = []
    o = 0
    while o < _CHUNK:
        n = min(_SUB, _CHUNK - o)
        subs.append((o, n))
        o += n
    mesh = plsc.VectorSubcoreMesh(core_axis_name="c", subcore_axis_name="s")

    @functools.partial(
        pl.kernel,
        out_type=jax.ShapeDtypeStruct((n_total, EMB_D), jnp.float32),
        mesh=mesh,
        compiler_params=pltpu.CompilerParams(use_tc_tiling_on_sc=True),
        scratch_types=[
            pltpu.VMEM((_CHUNK,), jnp.int32),
            pltpu.VMEM((_CHUNK,), jnp.int32),
            pltpu.VMEM((_CHUNK, EMB_D), jnp.float32),
            pltpu.VMEM((_CHUNK, EMB_D), jnp.float32),
            pltpu.SemaphoreType.DMA,
            pltpu.SemaphoreType.DMA,
            pltpu.SemaphoreType.DMA,
            pltpu.SemaphoreType.DMA,
            pltpu.SemaphoreType.DMA,
            pltpu.SemaphoreType.DMA,
        ],
    )
    def gather_kernel(idx_hbm, table_hbm, out_hbm, idx0, idx1, rows0, rows1,
                      isem0, isem1, gsem0, gsem1, ssem0, ssem1):
        wid = lax.axis_index("s") * _NC + lax.axis_index("c")
        base = wid * bpw
        slots = ((idx0, isem0, rows0, gsem0, ssem0),
                 (idx1, isem1, rows1, gsem1, ssem1))

        def start_idx(g, slot):
            idx_v, isem = slots[slot][0], slots[slot][1]
            pltpu.async_copy(idx_hbm.at[pl.ds(base + g * _CHUNK, _CHUNK)],
                             idx_v, isem)

        def fire_gather(g, slot, wait_store):
            idx_v, isem, rows_v, gsem, ssem = slots[slot]
            off = base + g * _CHUNK
            pltpu.make_async_copy(idx_hbm.at[pl.ds(off, _CHUNK)],
                                  idx_v, isem).wait()
            if wait_store:
                pltpu.make_async_copy(rows_v, out_hbm.at[pl.ds(off, _CHUNK)],
                                      ssem).wait()
            for (o, n) in subs:
                pltpu.async_copy(table_hbm.at[idx_v.at[pl.ds(o, n)]],
                                 rows_v.at[pl.ds(o, n)], gsem)

        def retire(g, slot, prefetch):
            idx_v, isem, rows_v, gsem, ssem = slots[slot]
            off = base + g * _CHUNK
            for (o, n) in subs:
                pltpu.make_async_copy(table_hbm.at[idx_v.at[pl.ds(o, n)]],
                                      rows_v.at[pl.ds(o, n)], gsem).wait()
            if prefetch:
                start_idx(g + 2, slot)
            pltpu.async_copy(rows_v, out_hbm.at[pl.ds(off, _CHUNK)], ssem)

        start_idx(0, 0)
        start_idx(1, 1)
        fire_gather(0, 0, wait_store=False)
        fire_gather(1, 1, wait_store=False)
        retire(0, 0, prefetch=True)

        def pair_body(k, carry):
            g = 2 + 2 * k
            fire_gather(g, 0, wait_store=True)
            retire(g - 1, 1, prefetch=True)
            fire_gather(g + 1, 1, wait_store=True)
            retire(g, 0, prefetch=True)
            return carry

        lax.fori_loop(0, (n_chunks - 4) // 2, pair_body, 0)
        g = n_chunks - 2
        fire_gather(g, 0, wait_store=True)
        retire(g - 1, 1, prefetch=True)
        fire_gather(g + 1, 1, wait_store=True)
        retire(g, 0, prefetch=False)
        retire(n_chunks - 1, 1, prefetch=False)
        pltpu.make_async_copy(rows0, out_hbm.at[pl.ds(base, _CHUNK)],
                              ssem0).wait()
        pltpu.make_async_copy(rows1, out_hbm.at[pl.ds(base, _CHUNK)],
                              ssem1).wait()

    return gather_kernel


def kernel(x, atom_emb):
    b, s = x.shape
    perm_idx = x.T.reshape(-1).astype(jnp.int32)
    out2d = _make_gather(b * s)(perm_idx, atom_emb)
    return out2d.reshape(s, b, EMB_D).transpose(1, 0, 2)
